# baseline (speedup 1.0000x reference)
import functools

import jax
import jax.numpy as jnp
from jax import lax
from jax.experimental import pallas as pl
from jax.experimental.pallas import tpu as pltpu
from jax.experimental.pallas import tpu_sc as plsc

_NC = 2
_NS = 16
_NW = _NC * _NS


_E_CHUNKS = 8


def _sc_gather(sym_emb_pad, syms_flat, syms_pad_f32, similar_idx, drugs_pad):
    BL = syms_flat.shape[0]
    B = similar_idx.shape[0]
    V, DP = sym_emb_pad.shape
    SP = syms_pad_f32.shape[1]
    ND = drugs_pad.shape[1]
    e_per = BL // _NW
    e_chunk = e_per // _E_CHUNKS
    b_per = B // _NW
    mesh = plsc.VectorSubcoreMesh(core_axis_name="c", subcore_axis_name="s")

    @functools.partial(
        pl.kernel,
        out_type=(
            jax.ShapeDtypeStruct((BL, DP), jnp.float32),
            jax.ShapeDtypeStruct((B, SP), jnp.float32),
            jax.ShapeDtypeStruct((B, ND), jnp.float32),
        ),
        mesh=mesh,
        scratch_types=[
            pltpu.VMEM((e_per,), jnp.int32),
            pltpu.VMEM((e_chunk, DP), jnp.float32),
            pltpu.VMEM((e_chunk, DP), jnp.float32),
            pltpu.VMEM((b_per,), jnp.int32),
            pltpu.VMEM((b_per, SP), jnp.float32),
            pltpu.VMEM((b_per, ND), jnp.float32),
            pltpu.SemaphoreType.DMA,
            pltpu.SemaphoreType.DMA,
            pltpu.SemaphoreType.DMA,
            pltpu.SemaphoreType.DMA,
            pltpu.VMEM_SHARED((V, DP), jnp.float32),
        ],
    )
    def gather_kernel(emb_hbm, sflat_hbm, spad_hbm, sim_hbm, drugs_hbm,
                      e_out, ssel_out, dsel_out,
                      idx_e, buf_e0, buf_e1, idx_b, buf_s, buf_d,
                      gsem, wsem, sem2, sem3, shared_tab):
        wid = lax.axis_index("s") * _NC + lax.axis_index("c")
        sid = lax.axis_index("s")
        eb = wid * e_per
        bb = wid * b_per
        bufs = (buf_e0, buf_e1)
        v_str = V // _NS
        pltpu.sync_copy(emb_hbm.at[pl.ds(sid * v_str, v_str)],
                        shared_tab.at[pl.ds(sid * v_str, v_str)])
        pltpu.sync_copy(sflat_hbm.at[pl.ds(eb, e_per)], idx_e)
        pltpu.sync_copy(sim_hbm.at[pl.ds(bb, b_per)], idx_b)
        c2 = pltpu.async_copy(spad_hbm.at[idx_b], buf_s, sem2)
        c3 = pltpu.async_copy(drugs_hbm.at[idx_b], buf_d, sem3)
        plsc.subcore_barrier()

        def g_start(t):
            return pltpu.async_copy(
                shared_tab.at[idx_e.at[pl.ds(t * e_chunk, e_chunk)]],
                bufs[t % 2], gsem)

        g = {0: g_start(0)}
        w = {}
        for t in range(_E_CHUNKS):
            if t + 1 < _E_CHUNKS:
                if t - 1 >= 0:
                    w[t - 1].wait()
                g[t + 1] = g_start(t + 1)
            g[t].wait()
            w[t] = pltpu.async_copy(
                bufs[t % 2], e_out.at[pl.ds(eb + t * e_chunk, e_chunk)], wsem)
        w[_E_CHUNKS - 2].wait()
        w[_E_CHUNKS - 1].wait()
        c2.wait()
        c3.wait()
        pltpu.sync_copy(buf_s, ssel_out.at[pl.ds(bb, b_per)])
        pltpu.sync_copy(buf_d, dsel_out.at[pl.ds(bb, b_per)])

    return gather_kernel(sym_emb_pad, syms_flat, syms_pad_f32, similar_idx,
                         drugs_pad)


def _sigmoid(x):
    return 1.0 / (1.0 + jnp.exp(-x))


def _tc_body(e_ref, syms_ref, ssel_ref, drugs_ref, dsel_ref,
             demb_ref, dembt_ref, ddi_ref, cw_ref,
             scores_ref, acc_ref, *, lv):
    bB, L, _ = e_ref.shape
    D = demb_ref.shape[1]

    valid = lax.broadcasted_iota(jnp.int32, (1, L, 1), 1) < lv
    u = jnp.where(valid, e_ref[...], 0.0)

    acc1 = jnp.sum(u, axis=1)
    s_set = acc1[:, :D] / acc1[:, D:D + 1]

    scores = lax.dot(s_set, dembt_ref[...])
    scores_ref[...] = scores

    P = _sigmoid(scores)
    ddi_part = jnp.sum(lax.dot(P, ddi_ref[...]) * P)

    s = syms_ref[...]
    ss = ssel_ref[...][:, :L]
    rhs = jnp.concatenate([ss, s], axis=1)
    eq = s[:, :, None] == rhs[:, None, :]
    cscore = jnp.sum(jnp.where(eq, cw_ref[...][None], 0.0), axis=2)
    c3 = (cscore > 0.0)[:, :, None]

    acc2 = jnp.sum(jnp.where(c3, u, 0.0), axis=1)
    z2 = acc2[:, D:D + 1]
    cset = acc2[:, :D] / jnp.where(z2 == 0.0, 1.0, z2)

    logits = lax.dot(cset, dembt_ref[...])
    drugs = drugs_ref[...]
    dsel = dsel_ref[...][:, :drugs.shape[1]]
    tgt = drugs * dsel
    loss = (jnp.maximum(logits, 0.0) - logits * tgt
            + jnp.log(1.0 + jnp.exp(-jnp.abs(logits))))
    bce_part = jnp.sum(loss)

    diff = drugs - dsel
    pos = jnp.maximum(diff, 0.0)
    neg = jnp.where(diff == -1.0, 1.0, 0.0)
    dd = jnp.sum(pos, axis=1, keepdims=True)
    dd2 = jnp.sum(neg, axis=1, keepdims=True)
    dd = jnp.where(dd == 0.0, 1.0, dd)
    dd2 = jnp.where(dd2 == 0.0, 1.0, dd2)
    de = lax.dot(pos, demb_ref[...]) / dd
    d2e = lax.dot(neg, demb_ref[...]) / dd2
    diff_part = jnp.sum(_sigmoid(cset * de) * _sigmoid(cset * d2e))

    lane1 = lax.broadcasted_iota(jnp.int32, (1, 128), 1)
    row = (jnp.where(lane1 == 0, bce_part, 0.0)
           + jnp.where(lane1 == 1, ddi_part, 0.0)
           + jnp.where(lane1 == 2, diff_part, 0.0))

    @pl.when(pl.program_id(0) == 0)
    def _():
        acc_ref[...] = jnp.zeros_like(acc_ref)

    acc_ref[...] += row


def _tc_compute(E3, syms_p, ssel, drugs, dsel_pad, demb, dembt,
                ddi, lv, *, interpret=False):
    B, L, DPE = E3.shape
    ND, D = demb.shape
    NDS = dsel_pad.shape[1]
    bB = 256
    grid = (B // bB,)

    ii = jnp.arange(L, dtype=jnp.int32)[:, None]
    jj = jnp.arange(2 * L, dtype=jnp.int32)[None, :]
    cw = jnp.where(jj < L, 1.0,
                   jnp.where((jj - L) < ii, -65536.0, 0.0)
                   ).astype(jnp.float32)

    return pl.pallas_call(
        functools.partial(_tc_body, lv=lv),
        grid=grid,
        in_specs=[
            pl.BlockSpec((bB, L, DPE), lambda i: (i, 0, 0)),
            pl.BlockSpec((bB, L), lambda i: (i, 0)),
            pl.BlockSpec((bB, ssel.shape[1]), lambda i: (i, 0)),
            pl.BlockSpec((bB, ND), lambda i: (i, 0)),
            pl.BlockSpec((bB, NDS), lambda i: (i, 0)),
            pl.BlockSpec((ND, D), lambda i: (0, 0)),
            pl.BlockSpec((D, ND), lambda i: (0, 0)),
            pl.BlockSpec((ND, ND), lambda i: (0, 0)),
            pl.BlockSpec((L, 2 * L), lambda i: (0, 0)),
        ],
        out_specs=[
            pl.BlockSpec((bB, ND), lambda i: (i, 0)),
            pl.BlockSpec((1, 128), lambda i: (0, 0)),
        ],
        out_shape=[
            jax.ShapeDtypeStruct((B, ND), jnp.float32),
            jax.ShapeDtypeStruct((1, 128), jnp.float32),
        ],
        interpret=interpret,
    )(E3, syms_p, ssel, drugs, dsel_pad, demb, dembt, ddi, cw)


def kernel(syms, drugs, similar_idx, sym_emb, drug_emb, agg_w, agg_b, ddi):
    B, L = syms.shape
    N_DRUG = drugs.shape[1]
    D = sym_emb.shape[1]
    NDP = 512
    SP = 128
    DP = 128
    LP = 56
    PAD_SYM = 2047
    NSYM_PAD = 2048

    syms = syms.astype(jnp.int32)
    similar_idx = similar_idx.astype(jnp.int32)

    n_sym = sym_emb.shape[0]
    q_tab = jnp.exp(jnp.tanh(sym_emb @ agg_w[:, 0] + agg_b[0]))
    rows = jnp.concatenate(
        [sym_emb * q_tab[:, None], q_tab[:, None],
         jnp.zeros((n_sym, DP - D - 1), jnp.float32)], axis=1)
    sym_emb_pad = jnp.pad(rows, ((0, NSYM_PAD - n_sym), (0, 0)))
    pad_fill = (jnp.arange(B * (LP - L), dtype=jnp.int32) % n_sym
                ).reshape(B, LP - L)
    syms_p = jnp.concatenate([syms, pad_fill], axis=1)
    syms_tab = jnp.concatenate(
        [syms, jnp.full((B, SP - L), PAD_SYM, jnp.int32)], axis=1)
    syms_tab_f32 = lax.bitcast_convert_type(syms_tab, jnp.float32)
    drugs_pad = jnp.pad(drugs, ((0, 0), (0, NDP - N_DRUG)))
    dembt = drug_emb.T
    syms_flat = syms_p.reshape(-1)

    e_flat, ssel_f32, dsel_pad = _sc_gather(
        sym_emb_pad, syms_flat, syms_tab_f32, similar_idx, drugs_pad)
    ssel = lax.bitcast_convert_type(ssel_f32, jnp.int32)
    E3 = e_flat.reshape(B, LP, DP)

    scores, acc = _tc_compute(
        E3, syms_p, ssel, drugs, dsel_pad, drug_emb, dembt, ddi, L)

    scores_aug = acc[0, 0] / jnp.float32(B * N_DRUG)
    batch_neg = acc[0, 1] * jnp.float32(1e-05) + acc[0, 2] * jnp.float32(1e-04)
    return scores, scores_aug, batch_neg
